# baseline (speedup 1.0000x reference)
import jax
import jax.numpy as jnp
from jax import lax
from jax.experimental import pallas as pl
from jax.experimental.pallas import tpu as pltpu
from jax.experimental.pallas import tpu_sc as plsc

_B, _N, _K = 128, 32768, 1024
_L = 16
_NCH = _N // _L
_NC, _NS = 2, 16
_NW = _NC * _NS
_RPW = _B // _NW

_MININT = -(2 ** 31)
_MAXINT = 2 ** 31 - 1


def _tokey(u):
    m = lax.shift_right_arithmetic(u, 31)
    return u ^ lax.shift_right_logical(m, 1)


def _clear(hist, nwords):
    z = jnp.zeros((_L,), jnp.int32)

    @plsc.parallel_loop(0, nwords // _L, unroll=8)
    def _(i):
        hist[pl.ds(i * _L, _L)] = z


def _keyify_scan(row, hist):
    ones = jnp.ones((_L,), jnp.int32)

    @plsc.parallel_loop(0, _NCH, unroll=8)
    def _(c):
        sl = pl.ds(c * _L, _L)
        ks = _tokey(plsc.bitcast(row[sl], jnp.int32))
        row[sl] = plsc.bitcast(ks, jnp.float32)
        b0 = lax.shift_right_logical(ks, 21) ^ 1024
        plsc.addupdate_scatter(hist, [b0], ones)


def _hist_scan(row, hist, get_bucket, get_mask, repr_ref=None):
    ones = jnp.ones((_L,), jnp.int32)

    @plsc.parallel_loop(0, _NCH, unroll=8)
    def _(c):
        ks = plsc.bitcast(row[pl.ds(c * _L, _L)], jnp.int32)
        b = get_bucket(ks)
        m = get_mask(ks)
        plsc.addupdate_scatter(hist, [b], ones, mask=m)
        if repr_ref is not None:
            plsc.store_scatter(repr_ref, [b], ks, mask=m)


def _find_seq(ref, nvals, target):
    nchk = nvals // _L

    def body(ii, carry):
        found, bkt, rank, ceq, run = carry
        base = (nchk - 1 - ii) * _L
        v = ref[pl.ds(base, _L)]
        S = lax.rev(jnp.cumsum(lax.rev(v, (0,))), (0,)) + run
        ge = S >= target
        hits = jnp.sum(ge.astype(jnp.int32))
        above = jnp.max(jnp.where(ge, run, S))
        smin = jnp.min(jnp.where(ge, S, _MAXINT))
        this = jnp.logical_and(found == 0, hits > 0)
        bkt = jnp.where(this, base + hits - 1, bkt)
        rank = jnp.where(this, target - above, rank)
        ceq = jnp.where(this, smin - above, ceq)
        found = jnp.where(hits > 0, jnp.int32(1), found)
        run = jnp.max(S)
        return found, bkt, rank, ceq, run

    init = tuple(jnp.int32(0) for _ in range(5))
    _, bkt, rank, ceq, _ = lax.fori_loop(0, nchk, body, init)
    return bkt, rank, ceq


def _find(hist, csum, nbuckets, target):
    nchk = nbuckets // _L
    lane0 = lax.iota(jnp.int32, _L) == 0

    @plsc.parallel_loop(0, nchk, unroll=4)
    def _(c):
        s = jnp.sum(hist[pl.ds(c * _L, _L)])
        plsc.store_scatter(csum, [jnp.full((_L,), c, jnp.int32)],
                           jnp.full((_L,), s, jnp.int32), mask=lane0)

    cc, t2, _ = _find_seq(csum, nchk, target)

    v = hist[pl.ds(cc * _L, _L)]
    S = lax.rev(jnp.cumsum(lax.rev(v, (0,))), (0,))
    ge = S >= t2
    hits = jnp.sum(ge.astype(jnp.int32))
    above = jnp.max(jnp.where(ge, 0, S))
    smin = jnp.min(jnp.where(ge, S, _MAXINT))
    return cc * _L + hits - 1, t2 - above, smin - above


def _mask_fast(row, t_s):
    @plsc.parallel_loop(0, _NCH, unroll=8)
    def _(i):
        sl = pl.ds(i * _L, _L)
        ks = plsc.bitcast(row[sl], jnp.int32)
        u = jnp.where(ks >= t_s, _tokey(ks), 0)
        row[sl] = plsc.bitcast(u, jnp.float32)


def _mask_slow(row, t_s, keep_eq):
    def body(i, eqc):
        sl = pl.ds(i * _L, _L)
        ks = plsc.bitcast(row[sl], jnp.int32)
        meq = ks == t_s
        cs = jnp.cumsum(meq.astype(jnp.int32)) + eqc
        keep = (ks > t_s) | (meq & (cs <= keep_eq))
        u = jnp.where(keep, _tokey(ks), 0)
        row[sl] = plsc.bitcast(u, jnp.float32)
        return eqc + jnp.sum(meq.astype(jnp.int32))

    lax.fori_loop(0, _NCH, body, jnp.int32(0))


def _process_row(row_v, hist_v, csum_v, repr_v):
    _clear(hist_v, 2048)
    _keyify_scan(row_v, hist_v)
    b0, j1, _ = _find(hist_v, csum_v, 2048, _K)
    b0k = b0 ^ 1024

    _clear(hist_v, 2048)
    _hist_scan(row_v, hist_v,
               lambda ks: lax.shift_right_logical(ks, 10) & 2047,
               lambda ks: lax.shift_right_logical(ks, 21) == b0k,
               repr_ref=repr_v)
    b1, j2, c1 = _find(hist_v, csum_v, 2048, j1)
    p01k = lax.shift_left(b0k, 11) | b1

    def unique_path(_):
        chunk = repr_v[pl.ds(b1 & -16, _L)]
        lane = b1 & 15
        iota = lax.iota(jnp.int32, _L)
        t_s = jnp.sum(jnp.where(iota == lane, chunk, 0))
        return t_s, jnp.int32(1), jnp.int32(1)

    def scan2_path(_):
        _clear(hist_v, 1024)
        _hist_scan(row_v, hist_v,
                   lambda ks: ks & 1023,
                   lambda ks: lax.shift_right_logical(ks, 10) == p01k)
        b2, j3, ceq = _find(hist_v, csum_v, 1024, j2)
        return lax.shift_left(p01k, 10) | b2, j3, ceq

    t_s, j3, ceq = lax.cond(c1 == 1, unique_path, scan2_path, 0)

    @pl.when(ceq == j3)
    def _():
        _mask_fast(row_v, t_s)

    @pl.when(ceq != j3)
    def _():
        _mask_slow(row_v, t_s, j3)


def _body(x_hbm, out_hbm, row0_v, row1_v, row2_v, hist_v, csum_v, repr_v,
          si0, si1, si2, so0, so1, so2):
    wid = lax.axis_index("s") * _NC + lax.axis_index("c")
    base_row = wid * _RPW
    bufs = (row0_v, row1_v, row2_v)
    sins = (si0, si1, si2)
    souts = (so0, so1, so2)

    def in_copy(r):
        b = r % 3
        return pltpu.make_async_copy(x_hbm.at[base_row + r], bufs[b], sins[b])

    def out_copy(r):
        b = r % 3
        return pltpu.make_async_copy(bufs[b], out_hbm.at[base_row + r],
                                     souts[b])

    in_copy(0).start()
    for r in range(_RPW):
        if r + 1 < _RPW:
            if r - 2 >= 0:
                out_copy(r - 2).wait()
            in_copy(r + 1).start()
        in_copy(r).wait()
        _process_row(bufs[r % 3], hist_v, csum_v, repr_v)
        out_copy(r).start()
    for r in range(max(0, _RPW - 3), _RPW):
        out_copy(r).wait()


@jax.jit
def kernel(x):
    mesh = plsc.VectorSubcoreMesh(core_axis_name="c", subcore_axis_name="s")
    f = pl.kernel(
        _body,
        out_type=jax.ShapeDtypeStruct((_B, _N), jnp.float32),
        mesh=mesh,
        scratch_types=[
            pltpu.VMEM((_N,), jnp.float32),
            pltpu.VMEM((_N,), jnp.float32),
            pltpu.VMEM((_N,), jnp.float32),
            pltpu.VMEM((2048,), jnp.int32),
            pltpu.VMEM((128,), jnp.int32),
            pltpu.VMEM((2048,), jnp.int32),
            pltpu.SemaphoreType.DMA,
            pltpu.SemaphoreType.DMA,
            pltpu.SemaphoreType.DMA,
            pltpu.SemaphoreType.DMA,
            pltpu.SemaphoreType.DMA,
            pltpu.SemaphoreType.DMA,
        ],
        compiler_params=pltpu.CompilerParams(needs_layout_passes=False),
    )
    return f(x)


# speedup vs baseline: 1.0017x; 1.0017x over previous
import jax
import jax.numpy as jnp
from jax import lax
from jax.experimental import pallas as pl
from jax.experimental.pallas import tpu as pltpu
from jax.experimental.pallas import tpu_sc as plsc

_B, _N, _K = 128, 32768, 1024
_L = 16
_NCH = _N // _L
_NC, _NS = 2, 16
_NW = _NC * _NS
_RPW = _B // _NW

_MAXINT = 2 ** 31 - 1


def _tokey(u):
    m = lax.shift_right_arithmetic(u, 31)
    return u ^ lax.shift_right_logical(m, 1)


def _clear(hist, nwords):
    z = jnp.zeros((_L,), jnp.int32)

    @plsc.parallel_loop(0, nwords // _L, unroll=8)
    def _(i):
        hist[pl.ds(i * _L, _L)] = z


def _keyify_scan(row, hist):
    ones = jnp.ones((_L,), jnp.int32)

    @plsc.parallel_loop(0, _NCH, unroll=8)
    def _(c):
        sl = pl.ds(c * _L, _L)
        ks = _tokey(plsc.bitcast(row[sl], jnp.int32))
        row[sl] = plsc.bitcast(ks, jnp.float32)
        b0 = lax.shift_right_logical(ks, 21) ^ 1024
        plsc.addupdate_scatter(hist, [b0], ones)


def _hist_scan(row, hist, get_bucket, get_mask, repr_ref=None):
    ones = jnp.ones((_L,), jnp.int32)

    @plsc.parallel_loop(0, _NCH, unroll=8)
    def _(c):
        ks = plsc.bitcast(row[pl.ds(c * _L, _L)], jnp.int32)
        b = get_bucket(ks)
        m = get_mask(ks)
        plsc.addupdate_scatter(hist, [b], ones, mask=m)
        if repr_ref is not None:
            plsc.store_scatter(repr_ref, [b], ks, mask=m)


def _find_seq(ref, nvals, target):
    nchk = nvals // _L

    def body(ii, carry):
        found, bkt, rank, ceq, run = carry
        base = (nchk - 1 - ii) * _L
        v = ref[pl.ds(base, _L)]
        S = lax.rev(jnp.cumsum(lax.rev(v, (0,))), (0,)) + run
        ge = S >= target
        hits = jnp.sum(ge.astype(jnp.int32))
        above = jnp.max(jnp.where(ge, run, S))
        smin = jnp.min(jnp.where(ge, S, _MAXINT))
        this = jnp.logical_and(found == 0, hits > 0)
        bkt = jnp.where(this, base + hits - 1, bkt)
        rank = jnp.where(this, target - above, rank)
        ceq = jnp.where(this, smin - above, ceq)
        found = jnp.where(hits > 0, jnp.int32(1), found)
        run = jnp.max(S)
        return found, bkt, rank, ceq, run

    init = tuple(jnp.int32(0) for _ in range(5))
    _, bkt, rank, ceq, _ = lax.fori_loop(0, nchk, body, init)
    return bkt, rank, ceq


def _find(hist, csum, nbuckets, target):
    nchk = nbuckets // _L
    lane0 = lax.iota(jnp.int32, _L) == 0

    @plsc.parallel_loop(0, nchk, unroll=4)
    def _(c):
        s = jnp.sum(hist[pl.ds(c * _L, _L)])
        plsc.store_scatter(csum, [jnp.full((_L,), c, jnp.int32)],
                           jnp.full((_L,), s, jnp.int32), mask=lane0)

    cc, t2, _ = _find_seq(csum, nchk, target)

    v = hist[pl.ds(cc * _L, _L)]
    S = lax.rev(jnp.cumsum(lax.rev(v, (0,))), (0,))
    ge = S >= t2
    hits = jnp.sum(ge.astype(jnp.int32))
    above = jnp.max(jnp.where(ge, 0, S))
    smin = jnp.min(jnp.where(ge, S, _MAXINT))
    return cc * _L + hits - 1, t2 - above, smin - above


def _mask_fast(row, t_s):
    @plsc.parallel_loop(0, _NCH, unroll=8)
    def _(i):
        sl = pl.ds(i * _L, _L)
        ks = plsc.bitcast(row[sl], jnp.int32)
        u = jnp.where(ks >= t_s, _tokey(ks), 0)
        row[sl] = plsc.bitcast(u, jnp.float32)


def _mask_slow(row, t_s, keep_eq):
    def body(i, eqc):
        sl = pl.ds(i * _L, _L)
        ks = plsc.bitcast(row[sl], jnp.int32)
        meq = ks == t_s
        cs = jnp.cumsum(meq.astype(jnp.int32)) + eqc
        keep = (ks > t_s) | (meq & (cs <= keep_eq))
        u = jnp.where(keep, _tokey(ks), 0)
        row[sl] = plsc.bitcast(u, jnp.float32)
        return eqc + jnp.sum(meq.astype(jnp.int32))

    lax.fori_loop(0, _NCH, body, jnp.int32(0))


def _process_row(row_v, hist_v, csum_v, repr_v):
    _clear(hist_v, 2048)
    _keyify_scan(row_v, hist_v)
    b0, j1, _ = _find(hist_v, csum_v, 2048, _K)
    b0k = b0 ^ 1024

    _clear(hist_v, 2048)
    _hist_scan(row_v, hist_v,
               lambda ks: lax.shift_right_logical(ks, 10) & 2047,
               lambda ks: lax.shift_right_logical(ks, 21) == b0k,
               repr_ref=repr_v)
    b1, j2, c1 = _find(hist_v, csum_v, 2048, j1)
    p01k = lax.shift_left(b0k, 11) | b1

    def unique_path(_):
        chunk = repr_v[pl.ds(b1 & -16, _L)]
        lane = b1 & 15
        iota = lax.iota(jnp.int32, _L)
        t_s = jnp.sum(jnp.where(iota == lane, chunk, 0))
        return t_s, jnp.int32(1), jnp.int32(1)

    def scan2_path(_):
        _clear(hist_v, 1024)
        _hist_scan(row_v, hist_v,
                   lambda ks: ks & 1023,
                   lambda ks: lax.shift_right_logical(ks, 10) == p01k)
        b2, j3, ceq = _find(hist_v, csum_v, 1024, j2)
        return lax.shift_left(p01k, 10) | b2, j3, ceq

    t_s, j3, ceq = lax.cond(c1 == 1, unique_path, scan2_path, 0)

    @pl.when(ceq == j3)
    def _():
        _mask_fast(row_v, t_s)

    @pl.when(ceq != j3)
    def _():
        _mask_slow(row_v, t_s, j3)


def _body(x_hbm, out_hbm, row0_v, row1_v, row2_v, hist_v, csum_v, repr_v,
          si0, si1, si2, so0, so1, so2):
    wid = lax.axis_index("s") * _NC + lax.axis_index("c")
    base_row = wid * _RPW
    bufs = (row0_v, row1_v, row2_v)
    sins = (si0, si1, si2)
    souts = (so0, so1, so2)

    def in_copy(r):
        b = r % 3
        return pltpu.make_async_copy(x_hbm.at[base_row + r], bufs[b], sins[b])

    def out_copy(r):
        b = r % 3
        return pltpu.make_async_copy(bufs[b], out_hbm.at[base_row + r],
                                     souts[b])

    in_copy(0).start()
    for r in range(_RPW):
        if r + 1 < _RPW:
            if r - 2 >= 0:
                out_copy(r - 2).wait()
            in_copy(r + 1).start()
        in_copy(r).wait()
        _process_row(bufs[r % 3], hist_v, csum_v, repr_v)
        out_copy(r).start()
    for r in range(max(0, _RPW - 3), _RPW):
        out_copy(r).wait()


@jax.jit
def kernel(x):
    mesh = plsc.VectorSubcoreMesh(core_axis_name="c", subcore_axis_name="s")
    f = pl.kernel(
        _body,
        out_type=jax.ShapeDtypeStruct((_B, _N), jnp.float32),
        mesh=mesh,
        scratch_types=[
            pltpu.VMEM((_N,), jnp.float32),
            pltpu.VMEM((_N,), jnp.float32),
            pltpu.VMEM((_N,), jnp.float32),
            pltpu.VMEM((2048,), jnp.int32),
            pltpu.VMEM((128,), jnp.int32),
            pltpu.VMEM((2048,), jnp.int32),
            pltpu.SemaphoreType.DMA,
            pltpu.SemaphoreType.DMA,
            pltpu.SemaphoreType.DMA,
            pltpu.SemaphoreType.DMA,
            pltpu.SemaphoreType.DMA,
            pltpu.SemaphoreType.DMA,
        ],
        compiler_params=pltpu.CompilerParams(needs_layout_passes=False),
    )
    return f(x)
